# baseline (speedup 1.0000x reference)
import functools

import jax
import jax.numpy as jnp
from jax import lax
from jax.experimental import pallas as pl
from jax.experimental.pallas import tpu as pltpu
from jax.experimental.pallas import tpu_sc as plsc

N_NODES = 10000
N_EDGES = 320000
IN_SIZE = 128
HID = 16
OUT = 64
H1 = 8
H2 = 1

NC = 2
NS = 16
NW = NC * NS
ROWB = 400
GRID = N_NODES // ROWB


def _make_sc_edge(n_nodes, n_edges, H, F):
  HF = H * F
  W = HF + 16
  EPW = n_edges // NW
  C = 80
  NCH = EPW // C
  G = C // 16
  RPT = n_nodes // NS
  NV = F // 16

  mesh = plsc.VectorSubcoreMesh(
      core_axis_name="c", subcore_axis_name="s", num_cores=NC,
      num_subcores=NS)

  @functools.partial(
      pl.kernel,
      out_type=jax.ShapeDtypeStruct((NC, n_nodes, W), jnp.float32),
      mesh=mesh,
      scratch_types=[
          pltpu.VMEM((2, C), jnp.int32),
          pltpu.VMEM((2, C), jnp.int32),
          pltpu.VMEM((2, C), jnp.int32),
          pltpu.VMEM((C,), jnp.int32),
          pltpu.VMEM((C,), jnp.int32),
          pltpu.VMEM((C,), jnp.int32),
          pltpu.VMEM((C, W), jnp.float32),
          pltpu.VMEM((C, W), jnp.float32),
          pltpu.VMEM((C, W), jnp.float32),
          pltpu.VMEM((C, 16), jnp.float32),
          pltpu.VMEM((C, 16), jnp.float32),
          pltpu.VMEM((C, 16), jnp.float32),
          pltpu.VMEM_SHARED((n_nodes, W), jnp.float32),
          pltpu.SemaphoreType.DMA,
          pltpu.SemaphoreType.DMA,
          pltpu.SemaphoreType.DMA,
          pltpu.SemaphoreType.DMA,
          pltpu.SemaphoreType.DMA,
          pltpu.SemaphoreType.DMA,
          pltpu.SemaphoreType.DMA,
          pltpu.SemaphoreType.DMA,
          pltpu.SemaphoreType.DMA,
      ],
      compiler_params=pltpu.CompilerParams(
          use_tc_tiling_on_sc=False, needs_layout_passes=False),
  )
  def sc_edge(ztab, ertab, eidx, zeros_h, out, ibuf0, ibuf1, ibuf2, dsc0,
              dsc1, dsc2, zbuf0, zbuf1, zbuf2, ebuf0, ebuf1, ebuf2, accum, isem0, isem1, isem2, gsem0, gsem1, gsem2, ssem0, ssem1,
              ssem2):
    cid = lax.axis_index("c")
    sid = lax.axis_index("s")
    wid = sid * NC + cid

    ibufs = [ibuf0, ibuf1, ibuf2]
    dscs = [dsc0, dsc1, dsc2]
    zbufs = [zbuf0, zbuf1, zbuf2]
    ebufs = [ebuf0, ebuf1, ebuf2]
    isems = [isem0, isem1, isem2]
    gsems = [gsem0, gsem1, gsem2]
    ssems = [ssem0, ssem1, ssem2]

    r0 = sid * RPT
    pltpu.sync_copy(zeros_h.at[pl.ds(r0, RPT)], accum.at[pl.ds(r0, RPT)])
    plsc.subcore_barrier()

    iota16 = lax.iota(jnp.int32, 16)

    def issue_idx(t, k):
      pltpu.async_copy(eidx.at[wid, t], ibufs[k], isems[k])

    def wait_idx(t, k):
      pltpu.make_async_copy(eidx.at[wid, t], ibufs[k], isems[k]).wait()

    def issue_gather(k):
      pltpu.async_copy(ztab.at[ibufs[k].at[0]], zbufs[k], gsems[k])
      pltpu.async_copy(ertab.at[ibufs[k].at[1]], ebufs[k], gsems[k])

    def wait_gather(k):
      pltpu.make_async_copy(ztab.at[ibufs[k].at[0]], zbufs[k], gsems[k]).wait()
      pltpu.make_async_copy(ertab.at[ibufs[k].at[1]], ebufs[k],
                            gsems[k]).wait()

    def copy_dst(k):
      for g in range(G):
        dscs[k][pl.ds(g * 16, 16)] = ibufs[k][1, pl.ds(g * 16, 16)]

    def issue_scatter(k):
      pltpu.async_copy(zbufs[k], accum.at[dscs[k]], ssems[k], add=True)

    def wait_scatter(k):
      pltpu.make_async_copy(zbufs[k], accum.at[dscs[k]], ssems[k]).wait()

    def process(zbuf, ebuf):
      def grp_body(g, carry2):
        eids = iota16 + g * 16
        exvs = []
        for h in range(H):
          el = plsc.load_gather(zbuf, [eids, jnp.full((16,), HF + h, jnp.int32)])
          er = plsc.load_gather(ebuf, [eids, jnp.full((16,), h, jnp.int32)])
          s = el + er
          e = jnp.where(s >= 0.0, s, 0.2 * s)
          exvs.append(jnp.exp(e))
        for h in range(H):
          plsc.store_scatter(zbuf, [eids, jnp.full((16,), HF + h, jnp.int32)],
                             exvs[h])
        for h in range(H):
          ex = exvs[h]
          for j in range(F):
            c = jnp.full((16,), h * F + j, jnp.int32)
            col = plsc.load_gather(zbuf, [eids, c])
            plsc.store_scatter(zbuf, [eids, c], col * ex)
        return carry2

      lax.fori_loop(0, G, grp_body, 0)

    def section(t, k, skip_swait=False):
      wait_gather(k)
      k2 = (k + 2) % 3

      @pl.when(t + 2 < NCH)
      def _():
        if not skip_swait:
          wait_scatter(k2)
        wait_idx(t + 2, k2)
        issue_gather(k2)

      copy_dst(k)

      @pl.when(t + 3 < NCH)
      def _():
        issue_idx(t + 3, k)

      process(zbufs[k], ebufs[k])
      issue_scatter(k)

    pltpu.sync_copy(eidx.at[wid, 0], ibuf0)
    pltpu.sync_copy(eidx.at[wid, 1], ibuf1)
    issue_gather(0)
    issue_gather(1)
    issue_idx(2, 2)

    section(0, 0, skip_swait=True)
    section(1, 1)
    section(2, 2)

    def pipe_body(u, carry):
      tA = 3 * u + 3
      section(tA, 0)
      section(tA + 1, 1)
      section(tA + 2, 2)
      return carry

    n_full = (NCH - 3) // 3
    lax.fori_loop(0, n_full, pipe_body, 0)
    for t in range(3 + 3 * n_full, NCH):
      section(t, t % 3)
    wait_scatter(0)
    wait_scatter(1)
    wait_scatter(2)
    plsc.subcore_barrier()
    pltpu.sync_copy(accum.at[pl.ds(r0, RPT)], out.at[cid, pl.ds(r0, RPT)])

  return sc_edge


_sc_edge_l1 = _make_sc_edge(N_NODES, N_EDGES, H1, HID)
_sc_edge_l2 = _make_sc_edge(N_NODES, N_EDGES, H2, OUT)


def _tc_pre1_body(x_ref, w_ref, al_ref, ar_ref, ztab_ref, ertab_ref):
  z = jnp.dot(x_ref[...], w_ref[...], preferred_element_type=jnp.float32)
  el = jnp.dot(z, al_ref[...], preferred_element_type=jnp.float32)
  er = jnp.dot(z, ar_ref[...], preferred_element_type=jnp.float32)
  pad = jnp.zeros((ROWB, 8), jnp.float32)
  ztab_ref[...] = jnp.concatenate([z, el, pad], axis=1)
  ertab_ref[...] = jnp.concatenate([er, pad], axis=1)


def _tc_mid_body(p_ref, w_ref, al_ref, ar_ref, r8_ref, ztab_ref, ertab_ref):
  p = p_ref[0] + p_ref[1]
  num = p[:, :IN_SIZE]
  den = p[:, IN_SIZE:IN_SIZE + H1]
  denw = jnp.dot(den, r8_ref[...], preferred_element_type=jnp.float32)
  h = num / (denw + 1e-9)
  h = jnp.where(h > 0.0, h, jnp.exp(h) - 1.0)
  z = jnp.dot(h, w_ref[...], preferred_element_type=jnp.float32)
  el = jnp.dot(z, al_ref[...], preferred_element_type=jnp.float32)
  er = jnp.dot(z, ar_ref[...], preferred_element_type=jnp.float32)
  pad = jnp.zeros((ROWB, 15), jnp.float32)
  ztab_ref[...] = jnp.concatenate([z, el, pad], axis=1)
  ertab_ref[...] = jnp.concatenate([er, pad], axis=1)


def _tc_post_body(p_ref, r1_ref, out_ref):
  p = p_ref[0] + p_ref[1]
  num = p[:, :OUT]
  den = p[:, OUT:OUT + 16]
  denw = jnp.dot(den, r1_ref[...], preferred_element_type=jnp.float32)
  out_ref[...] = num / (denw + 1e-9)


def kernel(x, edge_index, W1, aL1, aR1, W2, aL2, aR2):
  f32 = jnp.float32
  epw = N_EDGES // NW
  eidx = jnp.stack([
      edge_index[0].astype(jnp.int32).reshape(NW, epw // 80, 80),
      edge_index[1].astype(jnp.int32).reshape(NW, epw // 80, 80),
  ], axis=2)

  eye1 = jnp.eye(H1, dtype=f32)
  AL1 = (aL1[:, :, None] * eye1[:, None, :]).reshape(H1 * HID, H1)
  AR1 = (aR1[:, :, None] * eye1[:, None, :]).reshape(H1 * HID, H1)
  AL2 = jnp.transpose(aL2)
  AR2 = jnp.transpose(aR2)
  R8 = (jnp.arange(IN_SIZE)[None, :] // HID ==
        jnp.arange(H1)[:, None]).astype(f32)
  R1 = (jnp.arange(16)[:, None] == 0).astype(f32) * jnp.ones((16, OUT), f32)

  full = lambda shape: pl.BlockSpec(shape, lambda i: (0,) * len(shape))

  ztab1, ertab1 = pl.pallas_call(
      _tc_pre1_body,
      grid=(GRID,),
      in_specs=[
          pl.BlockSpec((ROWB, IN_SIZE), lambda i: (i, 0)),
          full((IN_SIZE, H1 * HID)),
          full((H1 * HID, H1)),
          full((H1 * HID, H1)),
      ],
      out_specs=[
          pl.BlockSpec((ROWB, IN_SIZE + 16), lambda i: (i, 0)),
          pl.BlockSpec((ROWB, 16), lambda i: (i, 0)),
      ],
      out_shape=[
          jax.ShapeDtypeStruct((N_NODES, IN_SIZE + 16), f32),
          jax.ShapeDtypeStruct((N_NODES, 16), f32),
      ],
  )(x, W1, AL1, AR1)

  zeros1 = jnp.zeros((N_NODES, IN_SIZE + 16), f32)
  parts1 = _sc_edge_l1(ztab1, ertab1, eidx, zeros1)

  ztab2, ertab2 = pl.pallas_call(
      _tc_mid_body,
      grid=(GRID,),
      in_specs=[
          pl.BlockSpec((NC, ROWB, IN_SIZE + 16), lambda i: (0, i, 0)),
          full((H1 * HID, H2 * OUT)),
          full((OUT, H2)),
          full((OUT, H2)),
          full((H1, IN_SIZE)),
      ],
      out_specs=[
          pl.BlockSpec((ROWB, OUT + 16), lambda i: (i, 0)),
          pl.BlockSpec((ROWB, 16), lambda i: (i, 0)),
      ],
      out_shape=[
          jax.ShapeDtypeStruct((N_NODES, OUT + 16), f32),
          jax.ShapeDtypeStruct((N_NODES, 16), f32),
      ],
  )(parts1, W2, AL2, AR2, R8)

  zeros2 = jnp.zeros((N_NODES, OUT + 16), f32)
  parts2 = _sc_edge_l2(ztab2, ertab2, eidx, zeros2)

  out = pl.pallas_call(
      _tc_post_body,
      grid=(GRID,),
      in_specs=[
          pl.BlockSpec((NC, ROWB, OUT + 16), lambda i: (0, i, 0)),
          full((16, OUT)),
      ],
      out_specs=pl.BlockSpec((ROWB, OUT), lambda i: (i, 0)),
      out_shape=jax.ShapeDtypeStruct((N_NODES, OUT), f32),
  )(parts2, R1)

  return out


# speedup vs baseline: 2.0494x; 2.0494x over previous
import functools

import numpy as np

import jax
import jax.numpy as jnp
from jax import lax
from jax.experimental import pallas as pl
from jax.experimental.pallas import tpu as pltpu
from jax.experimental.pallas import tpu_sc as plsc

N_NODES = 10000
N_EDGES = 320000
IN_SIZE = 128
HID = 16
OUT = 64
H1 = 8
H2 = 1

NC = 2
NS = 16
NW = NC * NS
ROWB = 400
GRID = N_NODES // ROWB


def _make_sc_edge(n_nodes, n_edges, H, F):
  HF = H * F
  W = HF + 16
  EPW = n_edges // NW
  C = 80
  NCH = EPW // C
  G = C // 16
  RPT = n_nodes // NS
  NV = F // 16

  mesh = plsc.VectorSubcoreMesh(
      core_axis_name="c", subcore_axis_name="s", num_cores=NC,
      num_subcores=NS)

  @functools.partial(
      pl.kernel,
      out_type=jax.ShapeDtypeStruct((NC, n_nodes, W), jnp.float32),
      mesh=mesh,
      scratch_types=[
          pltpu.VMEM((2, C), jnp.int32),
          pltpu.VMEM((2, C), jnp.int32),
          pltpu.VMEM((2, C), jnp.int32),
          pltpu.VMEM((C,), jnp.int32),
          pltpu.VMEM((C,), jnp.int32),
          pltpu.VMEM((C,), jnp.int32),
          pltpu.VMEM((C, W), jnp.float32),
          pltpu.VMEM((C, W), jnp.float32),
          pltpu.VMEM((C, W), jnp.float32),
          pltpu.VMEM((C, 16), jnp.float32),
          pltpu.VMEM((C, 16), jnp.float32),
          pltpu.VMEM((C, 16), jnp.float32),
          pltpu.VMEM_SHARED((n_nodes, W), jnp.float32),
          pltpu.SemaphoreType.DMA,
          pltpu.SemaphoreType.DMA,
          pltpu.SemaphoreType.DMA,
          pltpu.SemaphoreType.DMA,
          pltpu.SemaphoreType.DMA,
          pltpu.SemaphoreType.DMA,
          pltpu.SemaphoreType.DMA,
          pltpu.SemaphoreType.DMA,
          pltpu.SemaphoreType.DMA,
      ],
      compiler_params=pltpu.CompilerParams(
          use_tc_tiling_on_sc=False, needs_layout_passes=False),
  )
  def sc_edge(ztab, ertab, eidx, zeros_h, out, ibuf0, ibuf1, ibuf2, dsc0,
              dsc1, dsc2, zbuf0, zbuf1, zbuf2, ebuf0, ebuf1, ebuf2, accum, isem0, isem1, isem2, gsem0, gsem1, gsem2, ssem0, ssem1,
              ssem2):
    cid = lax.axis_index("c")
    sid = lax.axis_index("s")
    wid = sid * NC + cid

    ibufs = [ibuf0, ibuf1, ibuf2]
    dscs = [dsc0, dsc1, dsc2]
    zbufs = [zbuf0, zbuf1, zbuf2]
    ebufs = [ebuf0, ebuf1, ebuf2]
    isems = [isem0, isem1, isem2]
    gsems = [gsem0, gsem1, gsem2]
    ssems = [ssem0, ssem1, ssem2]

    r0 = sid * RPT
    pltpu.sync_copy(zeros_h.at[pl.ds(r0, RPT)], accum.at[pl.ds(r0, RPT)])
    plsc.subcore_barrier()

    iota16 = lax.iota(jnp.int32, 16)

    def issue_idx(t, k):
      pltpu.async_copy(eidx.at[wid, t], ibufs[k], isems[k])

    def wait_idx(t, k):
      pltpu.make_async_copy(eidx.at[wid, t], ibufs[k], isems[k]).wait()

    def issue_gather(k):
      pltpu.async_copy(ztab.at[ibufs[k].at[0]], zbufs[k], gsems[k])
      pltpu.async_copy(ertab.at[ibufs[k].at[1]], ebufs[k], gsems[k])

    def wait_gather(k):
      pltpu.make_async_copy(ztab.at[ibufs[k].at[0]], zbufs[k], gsems[k]).wait()
      pltpu.make_async_copy(ertab.at[ibufs[k].at[1]], ebufs[k],
                            gsems[k]).wait()

    def copy_dst(k):
      for g in range(G):
        dscs[k][pl.ds(g * 16, 16)] = ibufs[k][1, pl.ds(g * 16, 16)]

    def issue_scatter(k):
      pltpu.async_copy(zbufs[k], accum.at[dscs[k]], ssems[k], add=True)

    def wait_scatter(k):
      pltpu.make_async_copy(zbufs[k], accum.at[dscs[k]], ssems[k]).wait()

    maskh = iota16 < H

    def process(zbuf, ebuf):
      def edge_body(i, carry2):
        elv = zbuf[i, pl.ds(HF, 16)]
        erv = ebuf[i, pl.ds(0, 16)]
        s = elv + erv
        e = jnp.where(s >= 0.0, s, 0.2 * s)
        ex = jnp.where(maskh, jnp.exp(e), 0.0)
        zbuf[i, pl.ds(HF, 16)] = ex
        for h in range(H):
          sc = ex[h]
          for j in range(NV):
            off = h * F + j * 16
            zbuf[i, pl.ds(off, 16)] = zbuf[i, pl.ds(off, 16)] * sc
        return carry2

      lax.fori_loop(0, C, edge_body, 0)

    def section(t, k, skip_swait=False):
      wait_gather(k)
      k2 = (k + 2) % 3

      @pl.when(t + 2 < NCH)
      def _():
        if not skip_swait:
          wait_scatter(k2)
        wait_idx(t + 2, k2)
        issue_gather(k2)

      copy_dst(k)

      @pl.when(t + 3 < NCH)
      def _():
        issue_idx(t + 3, k)

      process(zbufs[k], ebufs[k])
      issue_scatter(k)

    pltpu.sync_copy(eidx.at[wid, 0], ibuf0)
    pltpu.sync_copy(eidx.at[wid, 1], ibuf1)
    issue_gather(0)
    issue_gather(1)
    issue_idx(2, 2)

    section(0, 0, skip_swait=True)
    section(1, 1)
    section(2, 2)

    def pipe_body(u, carry):
      tA = 3 * u + 3
      section(tA, 0)
      section(tA + 1, 1)
      section(tA + 2, 2)
      return carry

    n_full = (NCH - 3) // 3
    lax.fori_loop(0, n_full, pipe_body, 0)
    for t in range(3 + 3 * n_full, NCH):
      section(t, t % 3)
    wait_scatter(0)
    wait_scatter(1)
    wait_scatter(2)
    plsc.subcore_barrier()
    pltpu.sync_copy(accum.at[pl.ds(r0, RPT)], out.at[cid, pl.ds(r0, RPT)])

  return sc_edge


_sc_edge_l1 = _make_sc_edge(N_NODES, N_EDGES, H1, HID)
_sc_edge_l2 = _make_sc_edge(N_NODES, N_EDGES, H2, OUT)


def _tc_pre1_body(x_ref, w_ref, al_ref, ar_ref, ztab_ref, ertab_ref):
  z = jnp.dot(x_ref[...], w_ref[...], preferred_element_type=jnp.float32)
  el = jnp.dot(z, al_ref[...], preferred_element_type=jnp.float32)
  er = jnp.dot(z, ar_ref[...], preferred_element_type=jnp.float32)
  pad = jnp.zeros((ROWB, 8), jnp.float32)
  ztab_ref[...] = jnp.concatenate([z, el, pad], axis=1)
  ertab_ref[...] = jnp.concatenate([er, pad], axis=1)


def _tc_mid_body(p_ref, w_ref, al_ref, ar_ref, r8_ref, ztab_ref, ertab_ref):
  p = p_ref[0] + p_ref[1]
  num = p[:, :IN_SIZE]
  den = p[:, IN_SIZE:IN_SIZE + H1]
  denw = jnp.dot(den, r8_ref[...], preferred_element_type=jnp.float32)
  h = num / (denw + 1e-9)
  h = jnp.where(h > 0.0, h, jnp.exp(h) - 1.0)
  z = jnp.dot(h, w_ref[...], preferred_element_type=jnp.float32)
  el = jnp.dot(z, al_ref[...], preferred_element_type=jnp.float32)
  er = jnp.dot(z, ar_ref[...], preferred_element_type=jnp.float32)
  pad = jnp.zeros((ROWB, 15), jnp.float32)
  ztab_ref[...] = jnp.concatenate([z, el, pad], axis=1)
  ertab_ref[...] = jnp.concatenate([er, pad], axis=1)


def _tc_post_body(p_ref, r1_ref, out_ref):
  p = p_ref[0] + p_ref[1]
  num = p[:, :OUT]
  den = p[:, OUT:OUT + 16]
  denw = jnp.dot(den, r1_ref[...], preferred_element_type=jnp.float32)
  out_ref[...] = num / (denw + 1e-9)


def kernel(x, edge_index, W1, aL1, aR1, W2, aL2, aR2):
  f32 = jnp.float32
  epw = N_EDGES // NW
  eidx = jnp.stack([
      edge_index[0].astype(jnp.int32).reshape(NW, epw // 80, 80),
      edge_index[1].astype(jnp.int32).reshape(NW, epw // 80, 80),
  ], axis=2)

  eye1 = jnp.eye(H1, dtype=f32)
  AL1 = (aL1[:, :, None] * eye1[:, None, :]).reshape(H1 * HID, H1)
  AR1 = (aR1[:, :, None] * eye1[:, None, :]).reshape(H1 * HID, H1)
  AL2 = jnp.transpose(aL2)
  AR2 = jnp.transpose(aR2)
  R8 = (jnp.arange(IN_SIZE)[None, :] // HID ==
        jnp.arange(H1)[:, None]).astype(f32)
  R1 = (jnp.arange(16)[:, None] == 0).astype(f32) * jnp.ones((16, OUT), f32)

  full = lambda shape: pl.BlockSpec(shape, lambda i: (0,) * len(shape))

  ztab1, ertab1 = pl.pallas_call(
      _tc_pre1_body,
      grid=(GRID,),
      in_specs=[
          pl.BlockSpec((ROWB, IN_SIZE), lambda i: (i, 0)),
          full((IN_SIZE, H1 * HID)),
          full((H1 * HID, H1)),
          full((H1 * HID, H1)),
      ],
      out_specs=[
          pl.BlockSpec((ROWB, IN_SIZE + 16), lambda i: (i, 0)),
          pl.BlockSpec((ROWB, 16), lambda i: (i, 0)),
      ],
      out_shape=[
          jax.ShapeDtypeStruct((N_NODES, IN_SIZE + 16), f32),
          jax.ShapeDtypeStruct((N_NODES, 16), f32),
      ],
  )(x, W1, AL1, AR1)

  zeros1 = jnp.zeros((N_NODES, IN_SIZE + 16), f32)
  parts1 = _sc_edge_l1(ztab1, ertab1, eidx, zeros1)

  ztab2, ertab2 = pl.pallas_call(
      _tc_mid_body,
      grid=(GRID,),
      in_specs=[
          pl.BlockSpec((NC, ROWB, IN_SIZE + 16), lambda i: (0, i, 0)),
          full((H1 * HID, H2 * OUT)),
          full((OUT, H2)),
          full((OUT, H2)),
          full((H1, IN_SIZE)),
      ],
      out_specs=[
          pl.BlockSpec((ROWB, OUT + 16), lambda i: (i, 0)),
          pl.BlockSpec((ROWB, 16), lambda i: (i, 0)),
      ],
      out_shape=[
          jax.ShapeDtypeStruct((N_NODES, OUT + 16), f32),
          jax.ShapeDtypeStruct((N_NODES, 16), f32),
      ],
  )(parts1, W2, AL2, AR2, R8)

  zeros2 = jnp.zeros((N_NODES, OUT + 16), f32)
  parts2 = _sc_edge_l2(ztab2, ertab2, eidx, zeros2)

  out = pl.pallas_call(
      _tc_post_body,
      grid=(GRID,),
      in_specs=[
          pl.BlockSpec((NC, ROWB, OUT + 16), lambda i: (0, i, 0)),
          full((16, OUT)),
      ],
      out_specs=pl.BlockSpec((ROWB, OUT), lambda i: (i, 0)),
      out_shape=jax.ShapeDtypeStruct((N_NODES, OUT), f32),
  )(parts2, R1)

  return out


# speedup vs baseline: 2.3599x; 1.1515x over previous
import functools

import numpy as np

import jax
import jax.numpy as jnp
from jax import lax
from jax.experimental import pallas as pl
from jax.experimental.pallas import tpu as pltpu
from jax.experimental.pallas import tpu_sc as plsc

N_NODES = 10000
N_EDGES = 320000
IN_SIZE = 128
HID = 16
OUT = 64
H1 = 8
H2 = 1

NC = 2
NS = 16
NW = NC * NS
ROWB = 400
GRID = N_NODES // ROWB


def _make_sc_edge(n_nodes, n_edges, H, F):
  HF = H * F
  W = HF + 16
  EPW = n_edges // NW
  C = 80
  NCH = EPW // C
  G = C // 16
  RPT = n_nodes // NS
  NV = F // 16

  mesh = plsc.VectorSubcoreMesh(
      core_axis_name="c", subcore_axis_name="s", num_cores=NC,
      num_subcores=NS)

  @functools.partial(
      pl.kernel,
      out_type=jax.ShapeDtypeStruct((NC, n_nodes, W), jnp.float32),
      mesh=mesh,
      scratch_types=[
          pltpu.VMEM((2, C), jnp.int32),
          pltpu.VMEM((2, C), jnp.int32),
          pltpu.VMEM((2, C), jnp.int32),
          pltpu.VMEM((C,), jnp.int32),
          pltpu.VMEM((C,), jnp.int32),
          pltpu.VMEM((C,), jnp.int32),
          pltpu.VMEM((C, W), jnp.float32),
          pltpu.VMEM((C, W), jnp.float32),
          pltpu.VMEM((C, W), jnp.float32),
          pltpu.VMEM((C, 16), jnp.float32),
          pltpu.VMEM((C, 16), jnp.float32),
          pltpu.VMEM((C, 16), jnp.float32),
          pltpu.VMEM_SHARED((n_nodes, W), jnp.float32),
          pltpu.SemaphoreType.DMA,
          pltpu.SemaphoreType.DMA,
          pltpu.SemaphoreType.DMA,
          pltpu.SemaphoreType.DMA,
          pltpu.SemaphoreType.DMA,
          pltpu.SemaphoreType.DMA,
          pltpu.SemaphoreType.DMA,
          pltpu.SemaphoreType.DMA,
          pltpu.SemaphoreType.DMA,
      ],
      compiler_params=pltpu.CompilerParams(
          use_tc_tiling_on_sc=False, needs_layout_passes=False),
  )
  def sc_edge(ztab, ertab, eidx, zeros_h, out, ibuf0, ibuf1, ibuf2, dsc0,
              dsc1, dsc2, zbuf0, zbuf1, zbuf2, ebuf0, ebuf1, ebuf2, accum, isem0, isem1, isem2, gsem0, gsem1, gsem2, ssem0, ssem1,
              ssem2):
    cid = lax.axis_index("c")
    sid = lax.axis_index("s")
    wid = sid * NC + cid

    ibufs = [ibuf0, ibuf1, ibuf2]
    dscs = [dsc0, dsc1, dsc2]
    zbufs = [zbuf0, zbuf1, zbuf2]
    ebufs = [ebuf0, ebuf1, ebuf2]
    isems = [isem0, isem1, isem2]
    gsems = [gsem0, gsem1, gsem2]
    ssems = [ssem0, ssem1, ssem2]

    r0 = sid * RPT
    pltpu.sync_copy(zeros_h.at[pl.ds(r0, RPT)], accum.at[pl.ds(r0, RPT)])
    plsc.subcore_barrier()

    iota16 = lax.iota(jnp.int32, 16)

    def issue_idx(t, k):
      pltpu.async_copy(eidx.at[wid, t], ibufs[k], isems[k])

    def wait_idx(t, k):
      pltpu.make_async_copy(eidx.at[wid, t], ibufs[k], isems[k]).wait()

    def issue_gather(k):
      pltpu.async_copy(ztab.at[ibufs[k].at[0]], zbufs[k], gsems[k])
      pltpu.async_copy(ertab.at[ibufs[k].at[1]], ebufs[k], gsems[k])

    def wait_gather(k):
      pltpu.make_async_copy(ztab.at[ibufs[k].at[0]], zbufs[k], gsems[k]).wait()
      pltpu.make_async_copy(ertab.at[ibufs[k].at[1]], ebufs[k],
                            gsems[k]).wait()

    def copy_dst(k):
      for g in range(G):
        dscs[k][pl.ds(g * 16, 16)] = ibufs[k][1, pl.ds(g * 16, 16)]

    def issue_scatter(k):
      pltpu.async_copy(zbufs[k], accum.at[dscs[k]], ssems[k], add=True)

    def wait_scatter(k):
      pltpu.make_async_copy(zbufs[k], accum.at[dscs[k]], ssems[k]).wait()

    maskh = iota16 < H

    def process(zbuf, ebuf):
      def edge_body(i, carry2):
        elv = zbuf[i, pl.ds(HF, 16)]
        erv = ebuf[i, pl.ds(0, 16)]
        s = elv + erv
        e = jnp.where(s >= 0.0, s, 0.2 * s)
        ex = jnp.where(maskh, jnp.exp(e), 0.0)
        zbuf[i, pl.ds(HF, 16)] = ex
        for h in range(H):
          sc = ex[h]
          for j in range(NV):
            off = h * F + j * 16
            zbuf[i, pl.ds(off, 16)] = zbuf[i, pl.ds(off, 16)] * sc
        return carry2

      lax.fori_loop(0, C, edge_body, 0)

    def section(t, k, skip_swait=False):
      wait_gather(k)
      k2 = (k + 2) % 3
      copy_dst(k)

      @pl.when(t + 3 < NCH)
      def _():
        issue_idx(t + 3, k)

      process(zbufs[k], ebufs[k])

      @pl.when(t + 2 < NCH)
      def _():
        if not skip_swait:
          wait_scatter(k2)
        wait_idx(t + 2, k2)
        issue_gather(k2)

      issue_scatter(k)

    pltpu.sync_copy(eidx.at[wid, 0], ibuf0)
    pltpu.sync_copy(eidx.at[wid, 1], ibuf1)
    issue_gather(0)
    issue_gather(1)
    issue_idx(2, 2)

    section(0, 0, skip_swait=True)
    section(1, 1)
    section(2, 2)

    def pipe_body(u, carry):
      tA = 3 * u + 3
      section(tA, 0)
      section(tA + 1, 1)
      section(tA + 2, 2)
      return carry

    n_full = (NCH - 3) // 3
    lax.fori_loop(0, n_full, pipe_body, 0)
    for t in range(3 + 3 * n_full, NCH):
      section(t, t % 3)
    wait_scatter(0)
    wait_scatter(1)
    wait_scatter(2)
    plsc.subcore_barrier()
    pltpu.sync_copy(accum.at[pl.ds(r0, RPT)], out.at[cid, pl.ds(r0, RPT)])

  return sc_edge


_sc_edge_l1 = _make_sc_edge(N_NODES, N_EDGES, H1, HID)
_sc_edge_l2 = _make_sc_edge(N_NODES, N_EDGES, H2, OUT)


def _tc_pre1_body(x_ref, w_ref, al_ref, ar_ref, ztab_ref, ertab_ref):
  z = jnp.dot(x_ref[...], w_ref[...], preferred_element_type=jnp.float32)
  el = jnp.dot(z, al_ref[...], preferred_element_type=jnp.float32)
  er = jnp.dot(z, ar_ref[...], preferred_element_type=jnp.float32)
  pad = jnp.zeros((ROWB, 8), jnp.float32)
  ztab_ref[...] = jnp.concatenate([z, el, pad], axis=1)
  ertab_ref[...] = jnp.concatenate([er, pad], axis=1)


def _tc_mid_body(p_ref, w_ref, al_ref, ar_ref, r8_ref, ztab_ref, ertab_ref):
  p = p_ref[0] + p_ref[1]
  num = p[:, :IN_SIZE]
  den = p[:, IN_SIZE:IN_SIZE + H1]
  denw = jnp.dot(den, r8_ref[...], preferred_element_type=jnp.float32)
  h = num / (denw + 1e-9)
  h = jnp.where(h > 0.0, h, jnp.exp(h) - 1.0)
  z = jnp.dot(h, w_ref[...], preferred_element_type=jnp.float32)
  el = jnp.dot(z, al_ref[...], preferred_element_type=jnp.float32)
  er = jnp.dot(z, ar_ref[...], preferred_element_type=jnp.float32)
  pad = jnp.zeros((ROWB, 15), jnp.float32)
  ztab_ref[...] = jnp.concatenate([z, el, pad], axis=1)
  ertab_ref[...] = jnp.concatenate([er, pad], axis=1)


def _tc_post_body(p_ref, r1_ref, out_ref):
  p = p_ref[0] + p_ref[1]
  num = p[:, :OUT]
  den = p[:, OUT:OUT + 16]
  denw = jnp.dot(den, r1_ref[...], preferred_element_type=jnp.float32)
  out_ref[...] = num / (denw + 1e-9)


def kernel(x, edge_index, W1, aL1, aR1, W2, aL2, aR2):
  f32 = jnp.float32
  epw = N_EDGES // NW
  eidx = jnp.stack([
      edge_index[0].astype(jnp.int32).reshape(NW, epw // 80, 80),
      edge_index[1].astype(jnp.int32).reshape(NW, epw // 80, 80),
  ], axis=2)

  eye1 = jnp.eye(H1, dtype=f32)
  AL1 = (aL1[:, :, None] * eye1[:, None, :]).reshape(H1 * HID, H1)
  AR1 = (aR1[:, :, None] * eye1[:, None, :]).reshape(H1 * HID, H1)
  AL2 = jnp.transpose(aL2)
  AR2 = jnp.transpose(aR2)
  R8 = (jnp.arange(IN_SIZE)[None, :] // HID ==
        jnp.arange(H1)[:, None]).astype(f32)
  R1 = (jnp.arange(16)[:, None] == 0).astype(f32) * jnp.ones((16, OUT), f32)

  full = lambda shape: pl.BlockSpec(shape, lambda i: (0,) * len(shape))

  ztab1, ertab1 = pl.pallas_call(
      _tc_pre1_body,
      grid=(GRID,),
      in_specs=[
          pl.BlockSpec((ROWB, IN_SIZE), lambda i: (i, 0)),
          full((IN_SIZE, H1 * HID)),
          full((H1 * HID, H1)),
          full((H1 * HID, H1)),
      ],
      out_specs=[
          pl.BlockSpec((ROWB, IN_SIZE + 16), lambda i: (i, 0)),
          pl.BlockSpec((ROWB, 16), lambda i: (i, 0)),
      ],
      out_shape=[
          jax.ShapeDtypeStruct((N_NODES, IN_SIZE + 16), f32),
          jax.ShapeDtypeStruct((N_NODES, 16), f32),
      ],
  )(x, W1, AL1, AR1)

  zeros1 = jnp.zeros((N_NODES, IN_SIZE + 16), f32)
  parts1 = _sc_edge_l1(ztab1, ertab1, eidx, zeros1)

  ztab2, ertab2 = pl.pallas_call(
      _tc_mid_body,
      grid=(GRID,),
      in_specs=[
          pl.BlockSpec((NC, ROWB, IN_SIZE + 16), lambda i: (0, i, 0)),
          full((H1 * HID, H2 * OUT)),
          full((OUT, H2)),
          full((OUT, H2)),
          full((H1, IN_SIZE)),
      ],
      out_specs=[
          pl.BlockSpec((ROWB, OUT + 16), lambda i: (i, 0)),
          pl.BlockSpec((ROWB, 16), lambda i: (i, 0)),
      ],
      out_shape=[
          jax.ShapeDtypeStruct((N_NODES, OUT + 16), f32),
          jax.ShapeDtypeStruct((N_NODES, 16), f32),
      ],
  )(parts1, W2, AL2, AR2, R8)

  zeros2 = jnp.zeros((N_NODES, OUT + 16), f32)
  parts2 = _sc_edge_l2(ztab2, ertab2, eidx, zeros2)

  out = pl.pallas_call(
      _tc_post_body,
      grid=(GRID,),
      in_specs=[
          pl.BlockSpec((NC, ROWB, OUT + 16), lambda i: (0, i, 0)),
          full((16, OUT)),
      ],
      out_specs=pl.BlockSpec((ROWB, OUT), lambda i: (i, 0)),
      out_shape=jax.ShapeDtypeStruct((N_NODES, OUT), f32),
  )(parts2, R1)

  return out
